# baseline (speedup 1.0000x reference)
import jax
import jax.numpy as jnp
from jax.experimental import pallas as pl
from jax.experimental.pallas import tpu as pltpu

_NFEAT = 256
_NHID = 64
_NHEADS = 2
_P = 2
_N = 4096
_H = _NHID * _NHEADS
_NC = _P * _NHEADS
_ALPHA = 0.2
_NEG = -9e15

_BM = 256
_BPROJ = 1024


def _proj_body(x_ref, wf_ref, a1_ref, a2_ref, whb_ref, f1_ref, f2_ref):
    x = x_ref[...]
    wh = jnp.dot(x, wf_ref[...], preferred_element_type=jnp.float32)
    whb_ref[...] = wh.astype(jnp.bfloat16)
    f1_ref[...] = jnp.dot(wh, a1_ref[...], preferred_element_type=jnp.float32)
    f2_ref[...] = jnp.dot(wh, a2_ref[...], preferred_element_type=jnp.float32)


def _attn_body(adj_ref, wh_ref, f1_ref, f2_ref, out_ref):
    adj = adj_ref[0]
    for j in range(2 * _NHEADS):
        s, h = divmod(j, _NHEADS)
        f1 = f1_ref[0, :, j:j + 1]
        f2 = f2_ref[0, j:j + 1, :]
        mb = f1 + jnp.max(f2)
        m = jnp.maximum(mb, _ALPHA * mb)
        e = f1 + f2
        e = jnp.maximum(e, _ALPHA * e)
        p = jnp.exp(e - m) * adj
        ssum = jnp.sum(p, axis=1, keepdims=True)
        wh = wh_ref[s * _N:(s + 1) * _N, h * _NHID:(h + 1) * _NHID]
        o = jnp.dot(p.astype(jnp.bfloat16), wh,
                    preferred_element_type=jnp.float32) / ssum
        o = jnp.where(o > 0, o, jnp.exp(jnp.minimum(o, 0.0)) - 1.0)
        out_ref[s, 0, :, h * _NHID:(h + 1) * _NHID] = o


def _head_body(x_ref, wsem_ref, bsem_ref, qsem_ref, msk_ref, dw_ref, db_ref,
               sb1_ref, sb2_ref, out_ref):
    wsem = wsem_ref[...]
    bsem = bsem_ref[...]
    qsem = qsem_ref[...]
    hs = []
    for s in range(2):
        x0 = x_ref[s, 0]
        x1 = x_ref[s, 1]
        t0 = jnp.tanh(jnp.dot(x0, wsem, preferred_element_type=jnp.float32) + bsem)
        t1 = jnp.tanh(jnp.dot(x1, wsem, preferred_element_type=jnp.float32) + bsem)
        sem0 = jnp.mean(jnp.dot(t0, qsem, preferred_element_type=jnp.float32))
        sem1 = jnp.mean(jnp.dot(t1, qsem, preferred_element_type=jnp.float32))
        mx = jnp.maximum(sem0, sem1)
        e0 = jnp.exp(sem0 - mx)
        e1 = jnp.exp(sem1 - mx)
        den = e0 + e1
        hs.append(x0 * (e0 / den) + x1 * (e1 / den))
    h1, h2 = hs
    msk = msk_ref[...]
    c = jnp.sum(h1 * msk, axis=0, keepdims=True) / jnp.sum(msk)
    c = jax.nn.sigmoid(c)
    v = jnp.sum(dw_ref[...] * c, axis=1, keepdims=True)
    db = db_ref[0, 0]
    out_ref[:, 0:1] = jnp.dot(h1, v, preferred_element_type=jnp.float32) + db + sb1_ref[...]
    out_ref[:, 1:2] = jnp.dot(h2, v, preferred_element_type=jnp.float32) + db + sb2_ref[...]


def kernel(seq1, seq2, adjs, sparse, msk, samp_bias1, samp_bias2, W, a,
           Wsem, bsem, qsem, disc_W, disc_b):
    x2 = jnp.concatenate([seq1[0], seq2[0]], axis=0)
    wflat = jnp.transpose(W.reshape(_NC, _NFEAT, _NHID), (1, 0, 2)
                          ).reshape(_NFEAT, _NC * _NHID)
    a1 = a[..., :_NHID].reshape(_NC, _NHID)
    a2 = a[..., _NHID:].reshape(_NC, _NHID)
    eye = jnp.eye(_NC, dtype=jnp.float32)
    a1bd = (eye[:, None, :] * a1[:, :, None]).reshape(_NC * _NHID, _NC)
    a2bd = (eye[:, None, :] * a2[:, :, None]).reshape(_NC * _NHID, _NC)

    whall_bf, f1, f2 = pl.pallas_call(
        _proj_body,
        grid=(2 * _N // _BPROJ,),
        in_specs=[
            pl.BlockSpec((_BPROJ, _NFEAT), lambda i: (i, 0)),
            pl.BlockSpec((_NFEAT, _NC * _NHID), lambda i: (0, 0)),
            pl.BlockSpec((_NC * _NHID, _NC), lambda i: (0, 0)),
            pl.BlockSpec((_NC * _NHID, _NC), lambda i: (0, 0)),
        ],
        out_specs=[
            pl.BlockSpec((_BPROJ, _NC * _NHID), lambda i: (i, 0)),
            pl.BlockSpec((_BPROJ, _NC), lambda i: (i, 0)),
            pl.BlockSpec((_BPROJ, _NC), lambda i: (i, 0)),
        ],
        out_shape=[
            jax.ShapeDtypeStruct((2 * _N, _NC * _NHID), jnp.bfloat16),
            jax.ShapeDtypeStruct((2 * _N, _NC), jnp.float32),
            jax.ShapeDtypeStruct((2 * _N, _NC), jnp.float32),
        ],
    )(x2, wflat, a1bd, a2bd)

    f1_pb = f1.reshape(2, _N, _P, _NHEADS).transpose(2, 1, 0, 3).reshape(_P, _N, 2 * _NHEADS)
    f2_pb = f2.reshape(2, _N, _P, _NHEADS).transpose(2, 0, 3, 1).reshape(_P, 2 * _NHEADS, _N)

    x_all = pl.pallas_call(
        _attn_body,
        grid=(_P, _N // _BM),
        in_specs=[
            pl.BlockSpec((1, _BM, _N), lambda p, i: (p, i, 0)),
            pl.BlockSpec((2 * _N, _H), lambda p, i: (0, p)),
            pl.BlockSpec((1, _BM, 2 * _NHEADS), lambda p, i: (p, i, 0)),
            pl.BlockSpec((1, 2 * _NHEADS, _N), lambda p, i: (p, 0, 0)),
        ],
        out_specs=pl.BlockSpec((2, 1, _BM, _H), lambda p, i: (0, p, i, 0)),
        out_shape=jax.ShapeDtypeStruct((2, _P, _N, _H), jnp.float32),
        compiler_params=pltpu.CompilerParams(
            dimension_semantics=("parallel", "parallel")),
    )(adjs, whall_bf, f1_pb, f2_pb)

    out2 = pl.pallas_call(
        _head_body,
        out_shape=jax.ShapeDtypeStruct((_N, 2), jnp.float32),
    )(x_all, Wsem, bsem.reshape(1, -1), qsem, msk.reshape(-1, 1),
      disc_W, disc_b.reshape(1, 1),
      samp_bias1.reshape(-1, 1), samp_bias2.reshape(-1, 1))

    return out2.T.reshape(1, 2 * _N)


# speedup vs baseline: 1.3161x; 1.3161x over previous
import jax
import jax.numpy as jnp
from jax.experimental import pallas as pl
from jax.experimental.pallas import tpu as pltpu

_NFEAT = 256
_NHID = 64
_NHEADS = 2
_P = 2
_N = 4096
_H = _NHID * _NHEADS
_NC = _P * _NHEADS
_ALPHA = 0.2
_NEG = -9e15

_BM = 256
_BPROJ = 1024


_LOG2E = 1.4426950408889634


def _proj_body(x_ref, wf_ref, a1_ref, a2_ref, whb_ref, f1_ref, f2_ref):
    x = x_ref[...]
    wh = jnp.dot(x, wf_ref[...], preferred_element_type=jnp.float32)
    whb = wh.astype(jnp.bfloat16)
    for pp in range(_P):
        for hh in range(_NHEADS):
            base = hh * _H
            whb_ref[pp, :, base:base + _NHID] = whb[:, (pp * _NHEADS + hh) * _NHID:
                                                    (pp * _NHEADS + hh + 1) * _NHID]
            whb_ref[pp, :, base + _NHID:base + _H] = jnp.concatenate(
                [jnp.ones((_BPROJ, 1), jnp.bfloat16),
                 jnp.zeros((_BPROJ, _NHID - 1), jnp.bfloat16)], axis=1)
    f1_ref[...] = jnp.dot(wh, a1_ref[...], preferred_element_type=jnp.float32)
    f2_ref[...] = jnp.dot(wh, a2_ref[...], preferred_element_type=jnp.float32)


def _attn_body(adj_ref, wh_ref, f1_ref, f2_ref, out_ref):
    adj = adj_ref[0]
    for j in range(2 * _NHEADS):
        s, h = divmod(j, _NHEADS)
        f1 = f1_ref[0, :, j:j + 1] * _LOG2E
        f2 = f2_ref[0, j:j + 1, :] * _LOG2E
        mb = jnp.max(f1) + jnp.max(f2)
        m = jnp.maximum(mb, _ALPHA * mb)
        bvec = f2 - m
        cvec = _ALPHA * f1
        dvec = _ALPHA * f2 - m
        l = jnp.maximum(f1 + bvec, cvec + dvec)
        p = (jnp.exp2(l) * adj).astype(jnp.bfloat16)
        wha = wh_ref[0, s * _N:(s + 1) * _N, h * _H:(h + 1) * _H]
        res = jnp.dot(p, wha, preferred_element_type=jnp.float32)
        o = res[:, :_NHID] / res[:, _NHID:_NHID + 1]
        o = jnp.where(o > 0, o, jnp.exp(jnp.minimum(o, 0.0)) - 1.0)
        out_ref[s, 0, :, h * _NHID:(h + 1) * _NHID] = o


def _head_body(x_ref, wsem_ref, bsem_ref, qsem_ref, msk_ref, dw_ref, db_ref,
               sb1_ref, sb2_ref, out_ref):
    wsem = wsem_ref[...]
    bsem = bsem_ref[...]
    qsem = qsem_ref[...]
    hs = []
    for s in range(2):
        x0 = x_ref[s, 0]
        x1 = x_ref[s, 1]
        t0 = jnp.tanh(jnp.dot(x0, wsem, preferred_element_type=jnp.float32) + bsem)
        t1 = jnp.tanh(jnp.dot(x1, wsem, preferred_element_type=jnp.float32) + bsem)
        sem0 = jnp.mean(jnp.dot(t0, qsem, preferred_element_type=jnp.float32))
        sem1 = jnp.mean(jnp.dot(t1, qsem, preferred_element_type=jnp.float32))
        mx = jnp.maximum(sem0, sem1)
        e0 = jnp.exp(sem0 - mx)
        e1 = jnp.exp(sem1 - mx)
        den = e0 + e1
        hs.append(x0 * (e0 / den) + x1 * (e1 / den))
    h1, h2 = hs
    msk = msk_ref[...]
    c = jnp.sum(h1 * msk, axis=0, keepdims=True) / jnp.sum(msk)
    c = jax.nn.sigmoid(c)
    v = jnp.sum(dw_ref[...] * c, axis=1, keepdims=True)
    db = db_ref[0, 0]
    out_ref[:, 0:1] = jnp.dot(h1, v, preferred_element_type=jnp.float32) + db + sb1_ref[...]
    out_ref[:, 1:2] = jnp.dot(h2, v, preferred_element_type=jnp.float32) + db + sb2_ref[...]


def kernel(seq1, seq2, adjs, sparse, msk, samp_bias1, samp_bias2, W, a,
           Wsem, bsem, qsem, disc_W, disc_b):
    x2 = jnp.concatenate([seq1[0], seq2[0]], axis=0)
    wflat = jnp.transpose(W.reshape(_NC, _NFEAT, _NHID), (1, 0, 2)
                          ).reshape(_NFEAT, _NC * _NHID)
    a1 = a[..., :_NHID].reshape(_NC, _NHID)
    a2 = a[..., _NHID:].reshape(_NC, _NHID)
    eye = jnp.eye(_NC, dtype=jnp.float32)
    a1bd = (eye[:, None, :] * a1[:, :, None]).reshape(_NC * _NHID, _NC)
    a2bd = (eye[:, None, :] * a2[:, :, None]).reshape(_NC * _NHID, _NC)

    whall_bf, f1, f2 = pl.pallas_call(
        _proj_body,
        grid=(2 * _N // _BPROJ,),
        in_specs=[
            pl.BlockSpec((_BPROJ, _NFEAT), lambda i: (i, 0)),
            pl.BlockSpec((_NFEAT, _NC * _NHID), lambda i: (0, 0)),
            pl.BlockSpec((_NC * _NHID, _NC), lambda i: (0, 0)),
            pl.BlockSpec((_NC * _NHID, _NC), lambda i: (0, 0)),
        ],
        out_specs=[
            pl.BlockSpec((_P, _BPROJ, _NHEADS * _H), lambda i: (0, i, 0)),
            pl.BlockSpec((_BPROJ, _NC), lambda i: (i, 0)),
            pl.BlockSpec((_BPROJ, _NC), lambda i: (i, 0)),
        ],
        out_shape=[
            jax.ShapeDtypeStruct((_P, 2 * _N, _NHEADS * _H), jnp.bfloat16),
            jax.ShapeDtypeStruct((2 * _N, _NC), jnp.float32),
            jax.ShapeDtypeStruct((2 * _N, _NC), jnp.float32),
        ],
    )(x2, wflat, a1bd, a2bd)

    f1_pb = f1.reshape(2, _N, _P, _NHEADS).transpose(2, 1, 0, 3).reshape(_P, _N, 2 * _NHEADS)
    f2_pb = f2.reshape(2, _N, _P, _NHEADS).transpose(2, 0, 3, 1).reshape(_P, 2 * _NHEADS, _N)

    x_all = pl.pallas_call(
        _attn_body,
        grid=(_P, _N // _BM),
        in_specs=[
            pl.BlockSpec((1, _BM, _N), lambda p, i: (p, i, 0)),
            pl.BlockSpec((1, 2 * _N, _NHEADS * _H), lambda p, i: (p, 0, 0)),
            pl.BlockSpec((1, _BM, 2 * _NHEADS), lambda p, i: (p, i, 0)),
            pl.BlockSpec((1, 2 * _NHEADS, _N), lambda p, i: (p, 0, 0)),
        ],
        out_specs=pl.BlockSpec((2, 1, _BM, _H), lambda p, i: (0, p, i, 0)),
        out_shape=jax.ShapeDtypeStruct((2, _P, _N, _H), jnp.float32),
        compiler_params=pltpu.CompilerParams(
            dimension_semantics=("parallel", "parallel")),
    )(adjs, whall_bf, f1_pb, f2_pb)

    out2 = pl.pallas_call(
        _head_body,
        out_shape=jax.ShapeDtypeStruct((_N, 2), jnp.float32),
    )(x_all, Wsem, bsem.reshape(1, -1), qsem, msk.reshape(-1, 1),
      disc_W, disc_b.reshape(1, 1),
      samp_bias1.reshape(-1, 1), samp_bias2.reshape(-1, 1))

    return out2.T.reshape(1, 2 * _N)


# speedup vs baseline: 1.3418x; 1.0196x over previous
import jax
import jax.numpy as jnp
from jax.experimental import pallas as pl
from jax.experimental.pallas import tpu as pltpu

_NFEAT = 256
_NHID = 64
_NHEADS = 2
_P = 2
_N = 4096
_H = _NHID * _NHEADS
_NC = _P * _NHEADS
_ALPHA = 0.2
_NEG = -9e15

_BM = 256
_BPROJ = 1024


_LOG2E = 1.4426950408889634


def _proj_body(x_ref, wf_ref, a1_ref, a2_ref, whb_ref, f1_ref, f2_ref,
               ef2_ref, ef2a_ref):
    x = x_ref[...]
    wh = jnp.dot(x, wf_ref[...], preferred_element_type=jnp.float32)
    whb = wh.astype(jnp.bfloat16)
    for pp in range(_P):
        for hh in range(_NHEADS):
            base = hh * _H
            whb_ref[pp, :, base:base + _NHID] = whb[:, (pp * _NHEADS + hh) * _NHID:
                                                    (pp * _NHEADS + hh + 1) * _NHID]
            whb_ref[pp, :, base + _NHID:base + _H] = jnp.concatenate(
                [jnp.ones((_BPROJ, 1), jnp.bfloat16),
                 jnp.zeros((_BPROJ, _NHID - 1), jnp.bfloat16)], axis=1)
    f2l = jnp.dot(wh, a2_ref[...], preferred_element_type=jnp.float32)
    f1_ref[...] = jnp.dot(wh, a1_ref[...], preferred_element_type=jnp.float32)
    f2_ref[...] = f2l
    ef2_ref[...] = jnp.exp2(f2l)
    ef2a_ref[...] = jnp.exp2(_ALPHA * f2l)


def _attn_body(adj_ref, wh_ref, f1_ref, f2_ref, ef2_ref, ef2a_ref, out_ref):
    adj = adj_ref[0]
    for j in range(2 * _NHEADS):
        s, h = divmod(j, _NHEADS)
        f1 = f1_ref[0, :, j:j + 1]
        f2 = f2_ref[0, j:j + 1, :]
        colb = ef2_ref[0, j:j + 1, :]
        cold = ef2a_ref[0, j:j + 1, :]
        mb = jnp.max(f1) + jnp.max(f2)
        m = jnp.maximum(mb, _ALPHA * mb)
        rowp = jnp.exp2(f1 - m)
        rowq = jnp.exp2(_ALPHA * f1 - m)
        p = (jnp.maximum(rowp * colb, rowq * cold) * adj).astype(jnp.bfloat16)
        wha = wh_ref[0, s * _N:(s + 1) * _N, h * _H:(h + 1) * _H]
        res = jnp.dot(p, wha, preferred_element_type=jnp.float32)
        o = res[:, :_NHID] / res[:, _NHID:_NHID + 1]
        o = jnp.where(o > 0, o, jnp.exp(jnp.minimum(o, 0.0)) - 1.0)
        out_ref[s, 0, :, h * _NHID:(h + 1) * _NHID] = o


def _head_body(x_ref, wsem_ref, bsem_ref, qsem_ref, msk_ref, dw_ref, db_ref,
               sb1_ref, sb2_ref, out_ref):
    wsem = wsem_ref[...]
    bsem = bsem_ref[...]
    qsem = qsem_ref[...]
    hs = []
    for s in range(2):
        x0 = x_ref[s, 0]
        x1 = x_ref[s, 1]
        t0 = jnp.tanh(jnp.dot(x0, wsem, preferred_element_type=jnp.float32) + bsem)
        t1 = jnp.tanh(jnp.dot(x1, wsem, preferred_element_type=jnp.float32) + bsem)
        sem0 = jnp.mean(jnp.dot(t0, qsem, preferred_element_type=jnp.float32))
        sem1 = jnp.mean(jnp.dot(t1, qsem, preferred_element_type=jnp.float32))
        mx = jnp.maximum(sem0, sem1)
        e0 = jnp.exp(sem0 - mx)
        e1 = jnp.exp(sem1 - mx)
        den = e0 + e1
        hs.append(x0 * (e0 / den) + x1 * (e1 / den))
    h1, h2 = hs
    msk = msk_ref[...]
    c = jnp.sum(h1 * msk, axis=0, keepdims=True) / jnp.sum(msk)
    c = jax.nn.sigmoid(c)
    v = jnp.sum(dw_ref[...] * c, axis=1, keepdims=True)
    db = db_ref[0, 0]
    out_ref[:, 0:1] = jnp.dot(h1, v, preferred_element_type=jnp.float32) + db + sb1_ref[...]
    out_ref[:, 1:2] = jnp.dot(h2, v, preferred_element_type=jnp.float32) + db + sb2_ref[...]


def kernel(seq1, seq2, adjs, sparse, msk, samp_bias1, samp_bias2, W, a,
           Wsem, bsem, qsem, disc_W, disc_b):
    x2 = jnp.concatenate([seq1[0], seq2[0]], axis=0)
    wflat = jnp.transpose(W.reshape(_NC, _NFEAT, _NHID), (1, 0, 2)
                          ).reshape(_NFEAT, _NC * _NHID)
    a1 = a[..., :_NHID].reshape(_NC, _NHID)
    a2 = a[..., _NHID:].reshape(_NC, _NHID)
    eye = jnp.eye(_NC, dtype=jnp.float32) * _LOG2E
    a1bd = (eye[:, None, :] * a1[:, :, None]).reshape(_NC * _NHID, _NC)
    a2bd = (eye[:, None, :] * a2[:, :, None]).reshape(_NC * _NHID, _NC)

    whaug, f1, f2, ef2, ef2a = pl.pallas_call(
        _proj_body,
        grid=(2 * _N // _BPROJ,),
        in_specs=[
            pl.BlockSpec((_BPROJ, _NFEAT), lambda i: (i, 0)),
            pl.BlockSpec((_NFEAT, _NC * _NHID), lambda i: (0, 0)),
            pl.BlockSpec((_NC * _NHID, _NC), lambda i: (0, 0)),
            pl.BlockSpec((_NC * _NHID, _NC), lambda i: (0, 0)),
        ],
        out_specs=[
            pl.BlockSpec((_P, _BPROJ, _NHEADS * _H), lambda i: (0, i, 0)),
            pl.BlockSpec((_BPROJ, _NC), lambda i: (i, 0)),
            pl.BlockSpec((_BPROJ, _NC), lambda i: (i, 0)),
            pl.BlockSpec((_BPROJ, _NC), lambda i: (i, 0)),
            pl.BlockSpec((_BPROJ, _NC), lambda i: (i, 0)),
        ],
        out_shape=[
            jax.ShapeDtypeStruct((_P, 2 * _N, _NHEADS * _H), jnp.bfloat16),
            jax.ShapeDtypeStruct((2 * _N, _NC), jnp.float32),
            jax.ShapeDtypeStruct((2 * _N, _NC), jnp.float32),
            jax.ShapeDtypeStruct((2 * _N, _NC), jnp.float32),
            jax.ShapeDtypeStruct((2 * _N, _NC), jnp.float32),
        ],
    )(x2, wflat, a1bd, a2bd)

    def _to_cols(v):
        return v.reshape(2, _N, _P, _NHEADS).transpose(2, 1, 0, 3).reshape(_P, _N, 2 * _NHEADS)

    def _to_rows(v):
        return v.reshape(2, _N, _P, _NHEADS).transpose(2, 0, 3, 1).reshape(_P, 2 * _NHEADS, _N)

    f1_pb = _to_cols(f1)
    f2_pb = _to_rows(f2)
    ef2_pb = _to_rows(ef2)
    ef2a_pb = _to_rows(ef2a)

    x_all = pl.pallas_call(
        _attn_body,
        grid=(_P, _N // _BM),
        in_specs=[
            pl.BlockSpec((1, _BM, _N), lambda p, i: (p, i, 0)),
            pl.BlockSpec((1, 2 * _N, _NHEADS * _H), lambda p, i: (p, 0, 0)),
            pl.BlockSpec((1, _BM, 2 * _NHEADS), lambda p, i: (p, i, 0)),
            pl.BlockSpec((1, 2 * _NHEADS, _N), lambda p, i: (p, 0, 0)),
            pl.BlockSpec((1, 2 * _NHEADS, _N), lambda p, i: (p, 0, 0)),
            pl.BlockSpec((1, 2 * _NHEADS, _N), lambda p, i: (p, 0, 0)),
        ],
        out_specs=pl.BlockSpec((2, 1, _BM, _H), lambda p, i: (0, p, i, 0)),
        out_shape=jax.ShapeDtypeStruct((2, _P, _N, _H), jnp.float32),
        compiler_params=pltpu.CompilerParams(
            dimension_semantics=("parallel", "parallel")),
    )(adjs, whaug, f1_pb, f2_pb, ef2_pb, ef2a_pb)

    out2 = pl.pallas_call(
        _head_body,
        out_shape=jax.ShapeDtypeStruct((_N, 2), jnp.float32),
    )(x_all, Wsem, bsem.reshape(1, -1), qsem, msk.reshape(-1, 1),
      disc_W, disc_b.reshape(1, 1),
      samp_bias1.reshape(-1, 1), samp_bias2.reshape(-1, 1))

    return out2.T.reshape(1, 2 * _N)


# speedup vs baseline: 1.5538x; 1.1580x over previous
import jax
import jax.numpy as jnp
from jax.experimental import pallas as pl
from jax.experimental.pallas import tpu as pltpu

_NFEAT = 256
_NHID = 64
_NHEADS = 2
_P = 2
_N = 4096
_H = _NHID * _NHEADS
_NC = _P * _NHEADS
_ALPHA = 0.2
_NEG = -9e15

_BM = 512
_BPROJ = 1024


_LOG2E = 1.4426950408889634


def _proj_body(x_ref, wf_ref, a1_ref, a2_ref, whb_ref, f1_ref, f2_ref,
               ef2_ref, ef2a_ref):
    x = x_ref[...]
    wh = jnp.dot(x, wf_ref[...], preferred_element_type=jnp.float32)
    whb = wh.astype(jnp.bfloat16)
    for pp in range(_P):
        for hh in range(_NHEADS):
            base = hh * _H
            whb_ref[pp, :, base:base + _NHID] = whb[:, (pp * _NHEADS + hh) * _NHID:
                                                    (pp * _NHEADS + hh + 1) * _NHID]
            whb_ref[pp, :, base + _NHID:base + _H] = jnp.concatenate(
                [jnp.ones((_BPROJ, 1), jnp.bfloat16),
                 jnp.zeros((_BPROJ, _NHID - 1), jnp.bfloat16)], axis=1)
    f2l = jnp.dot(wh, a2_ref[...], preferred_element_type=jnp.float32)
    f1_ref[...] = jnp.dot(wh, a1_ref[...], preferred_element_type=jnp.float32)
    f2_ref[...] = f2l
    ef2_ref[...] = jnp.exp2(f2l)
    ef2a_ref[...] = jnp.exp2(_ALPHA * f2l)


def _attn_body(adj_ref, wh_ref, f1_ref, f2_ref, ef2_ref, ef2a_ref, out_ref):
    adj = adj_ref[0]
    for j in range(2 * _NHEADS):
        s, h = divmod(j, _NHEADS)
        f1 = f1_ref[0, :, j:j + 1]
        f2 = f2_ref[0, j:j + 1, :]
        colb = ef2_ref[0, j:j + 1, :]
        cold = ef2a_ref[0, j:j + 1, :]
        mb = jnp.max(f1) + jnp.max(f2)
        m = jnp.maximum(mb, _ALPHA * mb)
        rowp = jnp.exp2(f1 - m)
        rowq = jnp.exp2(_ALPHA * f1 - m)
        p = (jnp.maximum(rowp * colb, rowq * cold) * adj).astype(jnp.bfloat16)
        wha = wh_ref[0, s * _N:(s + 1) * _N, h * _H:(h + 1) * _H]
        res = jnp.dot(p, wha, preferred_element_type=jnp.float32)
        o = res[:, :_NHID] / res[:, _NHID:_NHID + 1]
        o = jnp.where(o > 0, o, jnp.exp(jnp.minimum(o, 0.0)) - 1.0)
        out_ref[s, 0, :, h * _NHID:(h + 1) * _NHID] = o


def _head_body(x_ref, wsem_ref, bsem_ref, qsem_ref, msk_ref, dw_ref, db_ref,
               sb1_ref, sb2_ref, out_ref):
    wsem = wsem_ref[...]
    bsem = bsem_ref[...]
    qsem = qsem_ref[...]
    hs = []
    for s in range(2):
        x0 = x_ref[s, 0]
        x1 = x_ref[s, 1]
        t0 = jnp.tanh(jnp.dot(x0, wsem, preferred_element_type=jnp.float32) + bsem)
        t1 = jnp.tanh(jnp.dot(x1, wsem, preferred_element_type=jnp.float32) + bsem)
        sem0 = jnp.mean(jnp.dot(t0, qsem, preferred_element_type=jnp.float32))
        sem1 = jnp.mean(jnp.dot(t1, qsem, preferred_element_type=jnp.float32))
        mx = jnp.maximum(sem0, sem1)
        e0 = jnp.exp(sem0 - mx)
        e1 = jnp.exp(sem1 - mx)
        den = e0 + e1
        hs.append(x0 * (e0 / den) + x1 * (e1 / den))
    h1, h2 = hs
    msk = msk_ref[...]
    c = jnp.sum(h1 * msk, axis=0, keepdims=True) / jnp.sum(msk)
    c = jax.nn.sigmoid(c)
    v = jnp.sum(dw_ref[...] * c, axis=1, keepdims=True)
    db = db_ref[0, 0]
    out_ref[:, 0:1] = jnp.dot(h1, v, preferred_element_type=jnp.float32) + db + sb1_ref[...]
    out_ref[:, 1:2] = jnp.dot(h2, v, preferred_element_type=jnp.float32) + db + sb2_ref[...]


def kernel(seq1, seq2, adjs, sparse, msk, samp_bias1, samp_bias2, W, a,
           Wsem, bsem, qsem, disc_W, disc_b):
    x2 = jnp.concatenate([seq1[0], seq2[0]], axis=0)
    wflat = jnp.transpose(W.reshape(_NC, _NFEAT, _NHID), (1, 0, 2)
                          ).reshape(_NFEAT, _NC * _NHID)
    a1 = a[..., :_NHID].reshape(_NC, _NHID)
    a2 = a[..., _NHID:].reshape(_NC, _NHID)
    eye = jnp.eye(_NC, dtype=jnp.float32) * _LOG2E
    a1bd = (eye[:, None, :] * a1[:, :, None]).reshape(_NC * _NHID, _NC)
    a2bd = (eye[:, None, :] * a2[:, :, None]).reshape(_NC * _NHID, _NC)

    whaug, f1, f2, ef2, ef2a = pl.pallas_call(
        _proj_body,
        grid=(2 * _N // _BPROJ,),
        in_specs=[
            pl.BlockSpec((_BPROJ, _NFEAT), lambda i: (i, 0)),
            pl.BlockSpec((_NFEAT, _NC * _NHID), lambda i: (0, 0)),
            pl.BlockSpec((_NC * _NHID, _NC), lambda i: (0, 0)),
            pl.BlockSpec((_NC * _NHID, _NC), lambda i: (0, 0)),
        ],
        out_specs=[
            pl.BlockSpec((_P, _BPROJ, _NHEADS * _H), lambda i: (0, i, 0)),
            pl.BlockSpec((_BPROJ, _NC), lambda i: (i, 0)),
            pl.BlockSpec((_BPROJ, _NC), lambda i: (i, 0)),
            pl.BlockSpec((_BPROJ, _NC), lambda i: (i, 0)),
            pl.BlockSpec((_BPROJ, _NC), lambda i: (i, 0)),
        ],
        out_shape=[
            jax.ShapeDtypeStruct((_P, 2 * _N, _NHEADS * _H), jnp.bfloat16),
            jax.ShapeDtypeStruct((2 * _N, _NC), jnp.float32),
            jax.ShapeDtypeStruct((2 * _N, _NC), jnp.float32),
            jax.ShapeDtypeStruct((2 * _N, _NC), jnp.float32),
            jax.ShapeDtypeStruct((2 * _N, _NC), jnp.float32),
        ],
    )(x2, wflat, a1bd, a2bd)

    def _to_cols(v):
        return v.reshape(2, _N, _P, _NHEADS).transpose(2, 1, 0, 3).reshape(_P, _N, 2 * _NHEADS)

    def _to_rows(v):
        return v.reshape(2, _N, _P, _NHEADS).transpose(2, 0, 3, 1).reshape(_P, 2 * _NHEADS, _N)

    f1_pb = _to_cols(f1)
    f2_pb = _to_rows(f2)
    ef2_pb = _to_rows(ef2)
    ef2a_pb = _to_rows(ef2a)

    x_all = pl.pallas_call(
        _attn_body,
        grid=(_P, _N // _BM),
        in_specs=[
            pl.BlockSpec((1, _BM, _N), lambda p, i: (p, i, 0)),
            pl.BlockSpec((1, 2 * _N, _NHEADS * _H), lambda p, i: (p, 0, 0)),
            pl.BlockSpec((1, _BM, 2 * _NHEADS), lambda p, i: (p, i, 0)),
            pl.BlockSpec((1, 2 * _NHEADS, _N), lambda p, i: (p, 0, 0)),
            pl.BlockSpec((1, 2 * _NHEADS, _N), lambda p, i: (p, 0, 0)),
            pl.BlockSpec((1, 2 * _NHEADS, _N), lambda p, i: (p, 0, 0)),
        ],
        out_specs=pl.BlockSpec((2, 1, _BM, _H), lambda p, i: (0, p, i, 0)),
        out_shape=jax.ShapeDtypeStruct((2, _P, _N, _H), jnp.float32),
        compiler_params=pltpu.CompilerParams(
            dimension_semantics=("parallel", "parallel")),
    )(adjs, whaug, f1_pb, f2_pb, ef2_pb, ef2a_pb)

    out2 = pl.pallas_call(
        _head_body,
        out_shape=jax.ShapeDtypeStruct((_N, 2), jnp.float32),
    )(x_all, Wsem, bsem.reshape(1, -1), qsem, msk.reshape(-1, 1),
      disc_W, disc_b.reshape(1, 1),
      samp_bias1.reshape(-1, 1), samp_bias2.reshape(-1, 1))

    return out2.T.reshape(1, 2 * _N)


# speedup vs baseline: 1.5587x; 1.0032x over previous
import jax
import jax.numpy as jnp
from jax.experimental import pallas as pl
from jax.experimental.pallas import tpu as pltpu

_NFEAT = 256
_NHID = 64
_NHEADS = 2
_P = 2
_N = 4096
_H = _NHID * _NHEADS
_NC = _P * _NHEADS
_ALPHA = 0.2
_NEG = -9e15

_BM = 1024
_BPROJ = 1024


_LOG2E = 1.4426950408889634


def _proj_body(x_ref, wf_ref, a1_ref, a2_ref, whb_ref, f1_ref, f2_ref,
               ef2_ref, ef2a_ref):
    x = x_ref[...]
    wh = jnp.dot(x, wf_ref[...], preferred_element_type=jnp.float32)
    whb = wh.astype(jnp.bfloat16)
    for pp in range(_P):
        for hh in range(_NHEADS):
            base = hh * _H
            whb_ref[pp, :, base:base + _NHID] = whb[:, (pp * _NHEADS + hh) * _NHID:
                                                    (pp * _NHEADS + hh + 1) * _NHID]
            whb_ref[pp, :, base + _NHID:base + _H] = jnp.concatenate(
                [jnp.ones((_BPROJ, 1), jnp.bfloat16),
                 jnp.zeros((_BPROJ, _NHID - 1), jnp.bfloat16)], axis=1)
    f2l = jnp.dot(wh, a2_ref[...], preferred_element_type=jnp.float32)
    f1_ref[...] = jnp.dot(wh, a1_ref[...], preferred_element_type=jnp.float32)
    f2_ref[...] = f2l
    ef2_ref[...] = jnp.exp2(f2l)
    ef2a_ref[...] = jnp.exp2(_ALPHA * f2l)


def _attn_body(adj_ref, wh_ref, f1_ref, f2_ref, ef2_ref, ef2a_ref, out_ref):
    adj = adj_ref[0]
    for j in range(2 * _NHEADS):
        s, h = divmod(j, _NHEADS)
        f1 = f1_ref[0, :, j:j + 1]
        f2 = f2_ref[0, j:j + 1, :]
        colb = ef2_ref[0, j:j + 1, :]
        cold = ef2a_ref[0, j:j + 1, :]
        mb = jnp.max(f1) + jnp.max(f2)
        m = jnp.maximum(mb, _ALPHA * mb)
        rowp = jnp.exp2(f1 - m)
        rowq = jnp.exp2(_ALPHA * f1 - m)
        p = (jnp.maximum(rowp * colb, rowq * cold) * adj).astype(jnp.bfloat16)
        wha = wh_ref[0, s * _N:(s + 1) * _N, h * _H:(h + 1) * _H]
        res = jnp.dot(p, wha, preferred_element_type=jnp.float32)
        o = res[:, :_NHID] / res[:, _NHID:_NHID + 1]
        o = jnp.where(o > 0, o, jnp.exp(jnp.minimum(o, 0.0)) - 1.0)
        out_ref[s, 0, :, h * _NHID:(h + 1) * _NHID] = o


def _head_body(x_ref, wsem_ref, bsem_ref, qsem_ref, msk_ref, dw_ref, db_ref,
               sb1_ref, sb2_ref, out_ref):
    wsem = wsem_ref[...]
    bsem = bsem_ref[...]
    qsem = qsem_ref[...]
    hs = []
    for s in range(2):
        x0 = x_ref[s, 0]
        x1 = x_ref[s, 1]
        t0 = jnp.tanh(jnp.dot(x0, wsem, preferred_element_type=jnp.float32) + bsem)
        t1 = jnp.tanh(jnp.dot(x1, wsem, preferred_element_type=jnp.float32) + bsem)
        sem0 = jnp.mean(jnp.dot(t0, qsem, preferred_element_type=jnp.float32))
        sem1 = jnp.mean(jnp.dot(t1, qsem, preferred_element_type=jnp.float32))
        mx = jnp.maximum(sem0, sem1)
        e0 = jnp.exp(sem0 - mx)
        e1 = jnp.exp(sem1 - mx)
        den = e0 + e1
        hs.append(x0 * (e0 / den) + x1 * (e1 / den))
    h1, h2 = hs
    msk = msk_ref[...]
    c = jnp.sum(h1 * msk, axis=0, keepdims=True) / jnp.sum(msk)
    c = jax.nn.sigmoid(c)
    v = jnp.sum(dw_ref[...] * c, axis=1, keepdims=True)
    db = db_ref[0, 0]
    out_ref[:, 0:1] = jnp.dot(h1, v, preferred_element_type=jnp.float32) + db + sb1_ref[...]
    out_ref[:, 1:2] = jnp.dot(h2, v, preferred_element_type=jnp.float32) + db + sb2_ref[...]


def kernel(seq1, seq2, adjs, sparse, msk, samp_bias1, samp_bias2, W, a,
           Wsem, bsem, qsem, disc_W, disc_b):
    x2 = jnp.concatenate([seq1[0], seq2[0]], axis=0)
    wflat = jnp.transpose(W.reshape(_NC, _NFEAT, _NHID), (1, 0, 2)
                          ).reshape(_NFEAT, _NC * _NHID)
    a1 = a[..., :_NHID].reshape(_NC, _NHID)
    a2 = a[..., _NHID:].reshape(_NC, _NHID)
    eye = jnp.eye(_NC, dtype=jnp.float32) * _LOG2E
    a1bd = (eye[:, None, :] * a1[:, :, None]).reshape(_NC * _NHID, _NC)
    a2bd = (eye[:, None, :] * a2[:, :, None]).reshape(_NC * _NHID, _NC)

    whaug, f1, f2, ef2, ef2a = pl.pallas_call(
        _proj_body,
        grid=(2 * _N // _BPROJ,),
        in_specs=[
            pl.BlockSpec((_BPROJ, _NFEAT), lambda i: (i, 0)),
            pl.BlockSpec((_NFEAT, _NC * _NHID), lambda i: (0, 0)),
            pl.BlockSpec((_NC * _NHID, _NC), lambda i: (0, 0)),
            pl.BlockSpec((_NC * _NHID, _NC), lambda i: (0, 0)),
        ],
        out_specs=[
            pl.BlockSpec((_P, _BPROJ, _NHEADS * _H), lambda i: (0, i, 0)),
            pl.BlockSpec((_BPROJ, _NC), lambda i: (i, 0)),
            pl.BlockSpec((_BPROJ, _NC), lambda i: (i, 0)),
            pl.BlockSpec((_BPROJ, _NC), lambda i: (i, 0)),
            pl.BlockSpec((_BPROJ, _NC), lambda i: (i, 0)),
        ],
        out_shape=[
            jax.ShapeDtypeStruct((_P, 2 * _N, _NHEADS * _H), jnp.bfloat16),
            jax.ShapeDtypeStruct((2 * _N, _NC), jnp.float32),
            jax.ShapeDtypeStruct((2 * _N, _NC), jnp.float32),
            jax.ShapeDtypeStruct((2 * _N, _NC), jnp.float32),
            jax.ShapeDtypeStruct((2 * _N, _NC), jnp.float32),
        ],
    )(x2, wflat, a1bd, a2bd)

    def _to_cols(v):
        return v.reshape(2, _N, _P, _NHEADS).transpose(2, 1, 0, 3).reshape(_P, _N, 2 * _NHEADS)

    def _to_rows(v):
        return v.reshape(2, _N, _P, _NHEADS).transpose(2, 0, 3, 1).reshape(_P, 2 * _NHEADS, _N)

    f1_pb = _to_cols(f1)
    f2_pb = _to_rows(f2)
    ef2_pb = _to_rows(ef2)
    ef2a_pb = _to_rows(ef2a)

    x_all = pl.pallas_call(
        _attn_body,
        grid=(_P, _N // _BM),
        in_specs=[
            pl.BlockSpec((1, _BM, _N), lambda p, i: (p, i, 0)),
            pl.BlockSpec((1, 2 * _N, _NHEADS * _H), lambda p, i: (p, 0, 0)),
            pl.BlockSpec((1, _BM, 2 * _NHEADS), lambda p, i: (p, i, 0)),
            pl.BlockSpec((1, 2 * _NHEADS, _N), lambda p, i: (p, 0, 0)),
            pl.BlockSpec((1, 2 * _NHEADS, _N), lambda p, i: (p, 0, 0)),
            pl.BlockSpec((1, 2 * _NHEADS, _N), lambda p, i: (p, 0, 0)),
        ],
        out_specs=pl.BlockSpec((2, 1, _BM, _H), lambda p, i: (0, p, i, 0)),
        out_shape=jax.ShapeDtypeStruct((2, _P, _N, _H), jnp.float32),
        compiler_params=pltpu.CompilerParams(
            dimension_semantics=("parallel", "parallel")),
    )(adjs, whaug, f1_pb, f2_pb, ef2_pb, ef2a_pb)

    out2 = pl.pallas_call(
        _head_body,
        out_shape=jax.ShapeDtypeStruct((_N, 2), jnp.float32),
    )(x_all, Wsem, bsem.reshape(1, -1), qsem, msk.reshape(-1, 1),
      disc_W, disc_b.reshape(1, 1),
      samp_bias1.reshape(-1, 1), samp_bias2.reshape(-1, 1))

    return out2.T.reshape(1, 2 * _N)


# speedup vs baseline: 2.0298x; 1.3023x over previous
import jax
import jax.numpy as jnp
from jax import lax
from jax.experimental import pallas as pl
from jax.experimental.pallas import tpu as pltpu

_NFEAT = 256
_NHID = 64
_NHEADS = 2
_P = 2
_N = 4096
_H = _NHID * _NHEADS
_NC = _P * _NHEADS
_ALPHA = 0.2
_LOG2E = 1.4426950408889634

_BM = 1024
_BPROJ = 1024


def _proj_body(s1_ref, s2_ref, wf_ref, a1_ref, a2_ref,
               whb_ref, f1_ref, f2_ref, ef2_ref, ef2a_ref):
    for s in range(2):
        x = s1_ref[0] if s == 0 else s2_ref[0]
        wh = jnp.dot(x, wf_ref[...], preferred_element_type=jnp.float32)
        whb = wh.astype(jnp.bfloat16)
        for pp in range(_P):
            for hh in range(_NHEADS):
                base = hh * _H
                c = pp * _NHEADS + hh
                whb_ref[pp, s, :, base:base + _NHID] = \
                    whb[:, c * _NHID:(c + 1) * _NHID]
                whb_ref[pp, s, :, base + _NHID:base + _H] = jnp.concatenate(
                    [jnp.ones((_BPROJ, 1), jnp.bfloat16),
                     jnp.zeros((_BPROJ, _NHID - 1), jnp.bfloat16)], axis=1)
        f1 = jnp.dot(wh, a1_ref[...], preferred_element_type=jnp.float32)
        f2r = lax.dot_general(a2_ref[...], wh, (((0,), (1,)), ((), ())),
                              preferred_element_type=jnp.float32)
        for pp in range(_P):
            lo, hi = s * _NHEADS, (s + 1) * _NHEADS
            clo, chi = pp * _NHEADS, (pp + 1) * _NHEADS
            f1_ref[pp, :, lo:hi] = f1[:, clo:chi]
            f2_ref[pp, lo:hi, :] = f2r[clo:chi, :]
            ef2_ref[pp, lo:hi, :] = jnp.exp2(f2r[clo:chi, :])
            ef2a_ref[pp, lo:hi, :] = jnp.exp2(_ALPHA * f2r[clo:chi, :])


def _attn_body(adj_ref, wh_ref, f1_ref, f2_ref, ef2_ref, ef2a_ref, out_ref):
    adj = adj_ref[0]
    for j in range(2 * _NHEADS):
        s, h = divmod(j, _NHEADS)
        f1 = f1_ref[0, :, j:j + 1]
        f2 = f2_ref[0, j:j + 1, :]
        colb = ef2_ref[0, j:j + 1, :]
        cold = ef2a_ref[0, j:j + 1, :]
        mb = jnp.max(f1) + jnp.max(f2)
        m = jnp.maximum(mb, _ALPHA * mb)
        rowp = jnp.exp2(f1 - m)
        rowq = jnp.exp2(_ALPHA * f1 - m)
        p = (jnp.maximum(rowp * colb, rowq * cold) * adj).astype(jnp.bfloat16)
        wha = wh_ref[0, s, :, h * _H:(h + 1) * _H]
        res = jnp.dot(p, wha, preferred_element_type=jnp.float32)
        o = res[:, :_NHID] / res[:, _NHID:_NHID + 1]
        o = jnp.where(o > 0, o, jnp.exp(jnp.minimum(o, 0.0)) - 1.0)
        out_ref[s, 0, :, h * _NHID:(h + 1) * _NHID] = o


def _head_body(x_ref, wsem_ref, bsem_ref, qsem_ref, msk_ref, dw_ref, db_ref,
               sb1_ref, sb2_ref, out_ref):
    wsem = wsem_ref[...]
    bsem = bsem_ref[...]
    qsem = qsem_ref[...]
    hs = []
    for s in range(2):
        x0 = x_ref[s, 0]
        x1 = x_ref[s, 1]
        t0 = jnp.tanh(jnp.dot(x0, wsem, preferred_element_type=jnp.float32) + bsem)
        t1 = jnp.tanh(jnp.dot(x1, wsem, preferred_element_type=jnp.float32) + bsem)
        sem0 = jnp.mean(jnp.dot(t0, qsem, preferred_element_type=jnp.float32))
        sem1 = jnp.mean(jnp.dot(t1, qsem, preferred_element_type=jnp.float32))
        mx = jnp.maximum(sem0, sem1)
        e0 = jnp.exp(sem0 - mx)
        e1 = jnp.exp(sem1 - mx)
        den = e0 + e1
        hs.append(x0 * (e0 / den) + x1 * (e1 / den))
    h1, h2 = hs
    msk = msk_ref[...]
    c = jnp.dot(msk, h1, preferred_element_type=jnp.float32) / jnp.sum(msk)
    c = jax.nn.sigmoid(c)
    v = lax.dot_general(c, dw_ref[...], (((1,), (1,)), ((), ())),
                        preferred_element_type=jnp.float32)
    sc1 = lax.dot_general(v, h1, (((1,), (1,)), ((), ())),
                          preferred_element_type=jnp.float32)
    sc2 = lax.dot_general(v, h2, (((1,), (1,)), ((), ())),
                          preferred_element_type=jnp.float32)
    db = db_ref[0]
    out_ref[0:1, 0:_N] = sc1 + db + sb1_ref[...]
    out_ref[0:1, _N:2 * _N] = sc2 + db + sb2_ref[...]


def kernel(seq1, seq2, adjs, sparse, msk, samp_bias1, samp_bias2, W, a,
           Wsem, bsem, qsem, disc_W, disc_b):
    wflat = jnp.transpose(W.reshape(_NC, _NFEAT, _NHID), (1, 0, 2)
                          ).reshape(_NFEAT, _NC * _NHID)
    a1 = a[..., :_NHID].reshape(_NC, _NHID)
    a2 = a[..., _NHID:].reshape(_NC, _NHID)
    eye = jnp.eye(_NC, dtype=jnp.float32) * _LOG2E
    a1bd = (eye[:, None, :] * a1[:, :, None]).reshape(_NC * _NHID, _NC)
    a2bd = (eye[:, None, :] * a2[:, :, None]).reshape(_NC * _NHID, _NC)

    whaug, f1_pb, f2_pb, ef2_pb, ef2a_pb = pl.pallas_call(
        _proj_body,
        grid=(_N // _BPROJ,),
        in_specs=[
            pl.BlockSpec((1, _BPROJ, _NFEAT), lambda i: (0, i, 0)),
            pl.BlockSpec((1, _BPROJ, _NFEAT), lambda i: (0, i, 0)),
            pl.BlockSpec((_NFEAT, _NC * _NHID), lambda i: (0, 0)),
            pl.BlockSpec((_NC * _NHID, _NC), lambda i: (0, 0)),
            pl.BlockSpec((_NC * _NHID, _NC), lambda i: (0, 0)),
        ],
        out_specs=[
            pl.BlockSpec((_P, 2, _BPROJ, _NHEADS * _H), lambda i: (0, 0, i, 0)),
            pl.BlockSpec((_P, _BPROJ, _NC), lambda i: (0, i, 0)),
            pl.BlockSpec((_P, _NC, _BPROJ), lambda i: (0, 0, i)),
            pl.BlockSpec((_P, _NC, _BPROJ), lambda i: (0, 0, i)),
            pl.BlockSpec((_P, _NC, _BPROJ), lambda i: (0, 0, i)),
        ],
        out_shape=[
            jax.ShapeDtypeStruct((_P, 2, _N, _NHEADS * _H), jnp.bfloat16),
            jax.ShapeDtypeStruct((_P, _N, _NC), jnp.float32),
            jax.ShapeDtypeStruct((_P, _NC, _N), jnp.float32),
            jax.ShapeDtypeStruct((_P, _NC, _N), jnp.float32),
            jax.ShapeDtypeStruct((_P, _NC, _N), jnp.float32),
        ],
    )(seq1, seq2, wflat, a1bd, a2bd)

    x_all = pl.pallas_call(
        _attn_body,
        grid=(_P, _N // _BM),
        in_specs=[
            pl.BlockSpec((1, _BM, _N), lambda p, i: (p, i, 0)),
            pl.BlockSpec((1, 2, _N, _NHEADS * _H), lambda p, i: (p, 0, 0, 0)),
            pl.BlockSpec((1, _BM, _NC), lambda p, i: (p, i, 0)),
            pl.BlockSpec((1, _NC, _N), lambda p, i: (p, 0, 0)),
            pl.BlockSpec((1, _NC, _N), lambda p, i: (p, 0, 0)),
            pl.BlockSpec((1, _NC, _N), lambda p, i: (p, 0, 0)),
        ],
        out_specs=pl.BlockSpec((2, 1, _BM, _H), lambda p, i: (0, p, i, 0)),
        out_shape=jax.ShapeDtypeStruct((2, _P, _N, _H), jnp.float32),
        compiler_params=pltpu.CompilerParams(
            dimension_semantics=("parallel", "parallel")),
    )(adjs, whaug, f1_pb, f2_pb, ef2_pb, ef2a_pb)

    return pl.pallas_call(
        _head_body,
        out_shape=jax.ShapeDtypeStruct((1, 2 * _N), jnp.float32),
    )(x_all, Wsem, bsem.reshape(1, -1), qsem, msk,
      disc_W, disc_b, samp_bias1, samp_bias2)


# speedup vs baseline: 2.1173x; 1.0431x over previous
import jax
import jax.numpy as jnp
from jax import lax
from jax.experimental import pallas as pl
from jax.experimental.pallas import tpu as pltpu

_NFEAT = 256
_NHID = 64
_NHEADS = 2
_P = 2
_N = 4096
_H = _NHID * _NHEADS
_NC = _P * _NHEADS
_ALPHA = 0.2
_LOG2E = 1.4426950408889634

_BM = 1024
_BPROJ = 1024


def _proj_body(s1_ref, s2_ref, wf_ref, a1_ref, a2_ref,
               whb_ref, f1_ref, f2_ref, ef2_ref, ef2a_ref):
    for s in range(2):
        x = s1_ref[0] if s == 0 else s2_ref[0]
        wh = jnp.dot(x, wf_ref[...], preferred_element_type=jnp.float32)
        whb = wh.astype(jnp.bfloat16)
        for pp in range(_P):
            for hh in range(_NHEADS):
                base = hh * _H
                c = pp * _NHEADS + hh
                whb_ref[pp, s, :, base:base + _NHID] = \
                    whb[:, c * _NHID:(c + 1) * _NHID]
                whb_ref[pp, s, :, base + _NHID:base + _H] = jnp.concatenate(
                    [jnp.ones((_BPROJ, 1), jnp.bfloat16),
                     jnp.zeros((_BPROJ, _NHID - 1), jnp.bfloat16)], axis=1)
        f1 = jnp.dot(wh, a1_ref[...], preferred_element_type=jnp.float32)
        f2r = lax.dot_general(a2_ref[...], wh, (((0,), (1,)), ((), ())),
                              preferred_element_type=jnp.float32)
        for pp in range(_P):
            lo, hi = s * _NHEADS, (s + 1) * _NHEADS
            clo, chi = pp * _NHEADS, (pp + 1) * _NHEADS
            f1_ref[pp, :, lo:hi] = f1[:, clo:chi]
            f2_ref[pp, lo:hi, :] = f2r[clo:chi, :]
            ef2_ref[pp, lo:hi, :] = jnp.exp2(f2r[clo:chi, :])
            ef2a_ref[pp, lo:hi, :] = jnp.exp2(_ALPHA * f2r[clo:chi, :])


def _attn_body(adj_ref, wh_ref, f1_ref, f2_ref, ef2_ref, ef2a_ref, out_ref):
    adjb = adj_ref[0].astype(jnp.bfloat16)
    for j in range(2 * _NHEADS):
        s, h = divmod(j, _NHEADS)
        f1 = f1_ref[0, :, j:j + 1]
        f2 = f2_ref[0, j:j + 1, :]
        colb = ef2_ref[0, j:j + 1, :]
        cold = ef2a_ref[0, j:j + 1, :]
        mb = jnp.max(f1) + jnp.max(f2)
        m = jnp.maximum(mb, _ALPHA * mb)
        rowp = jnp.exp2(f1 - m)
        rowq = jnp.exp2(_ALPHA * f1 - m)
        q = jnp.maximum(rowp * colb, rowq * cold)
        p = q.astype(jnp.bfloat16) * adjb
        wha = wh_ref[0, s, :, h * _H:(h + 1) * _H]
        res = jnp.dot(p, wha, preferred_element_type=jnp.float32)
        o = res[:, :_NHID] / res[:, _NHID:_NHID + 1]
        o = jnp.where(o > 0, o, jnp.exp(jnp.minimum(o, 0.0)) - 1.0)
        out_ref[s, 0, :, h * _NHID:(h + 1) * _NHID] = o


def _head_body(x_ref, wsem_ref, bsem_ref, qsem_ref, msk_ref, dw_ref, db_ref,
               sb1_ref, sb2_ref, out_ref):
    wsem = wsem_ref[...]
    bsem = bsem_ref[...]
    qsem = qsem_ref[...]
    hs = []
    for s in range(2):
        x0 = x_ref[s, 0]
        x1 = x_ref[s, 1]
        t0 = jnp.tanh(jnp.dot(x0, wsem, preferred_element_type=jnp.float32) + bsem)
        t1 = jnp.tanh(jnp.dot(x1, wsem, preferred_element_type=jnp.float32) + bsem)
        sem0 = jnp.mean(jnp.dot(t0, qsem, preferred_element_type=jnp.float32))
        sem1 = jnp.mean(jnp.dot(t1, qsem, preferred_element_type=jnp.float32))
        mx = jnp.maximum(sem0, sem1)
        e0 = jnp.exp(sem0 - mx)
        e1 = jnp.exp(sem1 - mx)
        den = e0 + e1
        hs.append(x0 * (e0 / den) + x1 * (e1 / den))
    h1, h2 = hs
    msk = msk_ref[...]
    c = jnp.dot(msk, h1, preferred_element_type=jnp.float32) / jnp.sum(msk)
    c = jax.nn.sigmoid(c)
    v = lax.dot_general(c, dw_ref[...], (((1,), (1,)), ((), ())),
                        preferred_element_type=jnp.float32)
    sc1 = lax.dot_general(v, h1, (((1,), (1,)), ((), ())),
                          preferred_element_type=jnp.float32)
    sc2 = lax.dot_general(v, h2, (((1,), (1,)), ((), ())),
                          preferred_element_type=jnp.float32)
    db = db_ref[0]
    out_ref[0:1, 0:_N] = sc1 + db + sb1_ref[...]
    out_ref[0:1, _N:2 * _N] = sc2 + db + sb2_ref[...]


def kernel(seq1, seq2, adjs, sparse, msk, samp_bias1, samp_bias2, W, a,
           Wsem, bsem, qsem, disc_W, disc_b):
    wflat = jnp.transpose(W.reshape(_NC, _NFEAT, _NHID), (1, 0, 2)
                          ).reshape(_NFEAT, _NC * _NHID)
    a1 = a[..., :_NHID].reshape(_NC, _NHID)
    a2 = a[..., _NHID:].reshape(_NC, _NHID)
    eye = jnp.eye(_NC, dtype=jnp.float32) * _LOG2E
    a1bd = (eye[:, None, :] * a1[:, :, None]).reshape(_NC * _NHID, _NC)
    a2bd = (eye[:, None, :] * a2[:, :, None]).reshape(_NC * _NHID, _NC)

    whaug, f1_pb, f2_pb, ef2_pb, ef2a_pb = pl.pallas_call(
        _proj_body,
        grid=(_N // _BPROJ,),
        in_specs=[
            pl.BlockSpec((1, _BPROJ, _NFEAT), lambda i: (0, i, 0)),
            pl.BlockSpec((1, _BPROJ, _NFEAT), lambda i: (0, i, 0)),
            pl.BlockSpec((_NFEAT, _NC * _NHID), lambda i: (0, 0)),
            pl.BlockSpec((_NC * _NHID, _NC), lambda i: (0, 0)),
            pl.BlockSpec((_NC * _NHID, _NC), lambda i: (0, 0)),
        ],
        out_specs=[
            pl.BlockSpec((_P, 2, _BPROJ, _NHEADS * _H), lambda i: (0, 0, i, 0)),
            pl.BlockSpec((_P, _BPROJ, _NC), lambda i: (0, i, 0)),
            pl.BlockSpec((_P, _NC, _BPROJ), lambda i: (0, 0, i)),
            pl.BlockSpec((_P, _NC, _BPROJ), lambda i: (0, 0, i)),
            pl.BlockSpec((_P, _NC, _BPROJ), lambda i: (0, 0, i)),
        ],
        out_shape=[
            jax.ShapeDtypeStruct((_P, 2, _N, _NHEADS * _H), jnp.bfloat16),
            jax.ShapeDtypeStruct((_P, _N, _NC), jnp.float32),
            jax.ShapeDtypeStruct((_P, _NC, _N), jnp.float32),
            jax.ShapeDtypeStruct((_P, _NC, _N), jnp.float32),
            jax.ShapeDtypeStruct((_P, _NC, _N), jnp.float32),
        ],
    )(seq1, seq2, wflat, a1bd, a2bd)

    x_all = pl.pallas_call(
        _attn_body,
        grid=(_P, _N // _BM),
        in_specs=[
            pl.BlockSpec((1, _BM, _N), lambda p, i: (p, i, 0)),
            pl.BlockSpec((1, 2, _N, _NHEADS * _H), lambda p, i: (p, 0, 0, 0)),
            pl.BlockSpec((1, _BM, _NC), lambda p, i: (p, i, 0)),
            pl.BlockSpec((1, _NC, _N), lambda p, i: (p, 0, 0)),
            pl.BlockSpec((1, _NC, _N), lambda p, i: (p, 0, 0)),
            pl.BlockSpec((1, _NC, _N), lambda p, i: (p, 0, 0)),
        ],
        out_specs=pl.BlockSpec((2, 1, _BM, _H), lambda p, i: (0, p, i, 0)),
        out_shape=jax.ShapeDtypeStruct((2, _P, _N, _H), jnp.float32),
        compiler_params=pltpu.CompilerParams(
            dimension_semantics=("parallel", "parallel")),
    )(adjs, whaug, f1_pb, f2_pb, ef2_pb, ef2a_pb)

    return pl.pallas_call(
        _head_body,
        out_shape=jax.ShapeDtypeStruct((1, 2 * _N), jnp.float32),
    )(x_all, Wsem, bsem.reshape(1, -1), qsem, msk,
      disc_W, disc_b, samp_bias1, samp_bias2)
